# baseline (speedup 1.0000x reference)
import functools

import jax
import jax.numpy as jnp
from jax import lax
from jax.experimental import pallas as pl
from jax.experimental.pallas import tpu as pltpu
from jax.experimental.pallas import tpu_sc as plsc

N = 10000
E = 320000
NP = 10240
NC, NS = 2, 16
NW = NC * NS
NCH, CH = 90, 112
EPT = NCH * CH
E_PAD = NW * EPT
RPS = NP // NS

_MESH = plsc.VectorSubcoreMesh(
    core_axis_name="c", subcore_axis_name="s", num_cores=NC, num_subcores=NS
)


def _deg_body(dst_hbm, ones_hbm, z1_hbm, degp_hbm, dst_v, ones_v, acc):
    c = lax.axis_index("c")
    s = lax.axis_index("s")
    wid = s * NC + c
    r0 = s * RPS
    pltpu.sync_copy(z1_hbm.at[pl.ds(r0, RPS)], acc.at[pl.ds(r0, RPS)])
    pltpu.sync_copy(dst_hbm.at[wid], dst_v)
    pltpu.sync_copy(ones_hbm, ones_v)
    plsc.subcore_barrier()

    def step(j, carry):
        pltpu.sync_copy(ones_v, acc.at[dst_v.at[j]], add=True)
        return carry

    lax.fori_loop(0, NCH, step, 0)
    plsc.subcore_barrier()
    pltpu.sync_copy(acc.at[pl.ds(r0, RPS)], degp_hbm.at[c, pl.ds(r0, RPS)])


_deg_kernel = functools.partial(
    pl.kernel,
    _deg_body,
    out_type=jax.ShapeDtypeStruct((NC, NP), jnp.float32),
    mesh=_MESH,
    scratch_types=[
        pltpu.VMEM((NCH, CH), jnp.int32),
        pltpu.VMEM((CH,), jnp.float32),
        pltpu.VMEM_SHARED((NP,), jnp.float32),
    ],
)()


def _make_agg(tail, dtype):
    def body(table, edges_hbm, z_hbm, out, *scr):
        idxs, rows, isems, gsems, acc = \
            scr[0:3], scr[3:6], scr[6:9], scr[9:12], scr[12]

        c = lax.axis_index("c")
        s = lax.axis_index("s")
        wid = s * NC + c
        r0 = s * RPS

        def idx_fetch(j, b):
            pltpu.async_copy(edges_hbm.at[wid, j], idxs[b], isems[b])

        def gather(j, b):
            pltpu.make_async_copy(edges_hbm.at[wid, j], idxs[b],
                                  isems[b]).wait()
            pltpu.async_copy(table.at[idxs[b].at[0]], rows[b], gsems[b])

        pltpu.sync_copy(z_hbm.at[pl.ds(r0, RPS)], acc.at[pl.ds(r0, RPS)])
        for b in range(3):
            idx_fetch(b, b)
        gather(0, 0)
        gather(1, 1)
        plsc.subcore_barrier()

        def group(g, carry):
            for b in range(3):
                j = g * 3 + b

                @pl.when(j + 2 < NCH)
                def _():
                    gather(j + 2, (b + 2) % 3)

                pltpu.make_async_copy(table.at[idxs[b].at[0]], rows[b],
                                      gsems[b]).wait()
                pltpu.sync_copy(rows[b], acc.at[idxs[b].at[1]], add=True)

                @pl.when(j + 3 < NCH)
                def _():
                    idx_fetch(j + 3, b)
            return carry

        lax.fori_loop(0, NCH // 3, group, 0)
        plsc.subcore_barrier()
        pltpu.sync_copy(acc.at[pl.ds(r0, RPS)], out.at[c, pl.ds(r0, RPS)])

    return functools.partial(
        pl.kernel,
        body,
        out_type=jax.ShapeDtypeStruct((NC, NP) + tail, dtype),
        mesh=_MESH,
        scratch_types=[
            *[pltpu.VMEM((2, CH), jnp.int32)] * 3,
            *[pltpu.VMEM((CH,) + tail, dtype)] * 3,
            *[pltpu.SemaphoreType.DMA] * 6,
            pltpu.VMEM_SHARED((NP,) + tail, dtype),
        ],
    )()


_agg128 = _make_agg((128,), jnp.float32)


_BM = 1280
_GRID = NP // _BM


def _mm1_body(x_ref, w_ref, o_ref):
    o_ref[...] = jnp.dot(x_ref[...], w_ref[...],
                         preferred_element_type=jnp.float32)


def _dinv(d0_ref, d1_ref):
    deg = d0_ref[...] + d1_ref[...] + 1.0
    return lax.rsqrt(jnp.maximum(deg, 1.0))


def _scale_body(t1_ref, d0_ref, d1_ref, a_ref, b_ref):
    dinv = _dinv(d0_ref, d1_ref)
    a_ref[...] = t1_ref[:, :128] * dinv
    b_ref[...] = t1_ref[:, 128:] * dinv


def _layer2_body(pa_ref, pb_ref, sa_ref, sb_ref, d0_ref, d1_ref, b1_ref,
                 w2_ref, o_ref):
    dinv = _dinv(d0_ref, d1_ref)
    ha = jax.nn.relu(dinv * (pa_ref[0] + pa_ref[1] + sa_ref[...])
                     + b1_ref[0:1, :128])
    hb = jax.nn.relu(dinv * (pb_ref[0] + pb_ref[1] + sb_ref[...])
                     + b1_ref[0:1, 128:])
    t2 = (jnp.dot(ha, w2_ref[:128, :], preferred_element_type=jnp.float32)
          + jnp.dot(hb, w2_ref[128:, :], preferred_element_type=jnp.float32))
    o_ref[...] = t2 * dinv


def _final_body(p_ref, s_ref, d0_ref, d1_ref, b2_ref, o_ref):
    agg = p_ref[0] + p_ref[1] + s_ref[...]
    o_ref[...] = _dinv(d0_ref, d1_ref) * agg + b2_ref[0:1, :]


def _row_spec(cols):
    return pl.BlockSpec((_BM, cols), lambda m: (m, 0))


def _whole_spec(shape):
    return pl.BlockSpec(shape, lambda m: tuple(0 for _ in shape))


def _part_spec(cols):
    return pl.BlockSpec((NC, _BM, cols), lambda m: (0, m, 0))


def kernel(x, edge_index, W1, b1, W2, b2):
    pad = (jnp.arange(E_PAD - E, dtype=jnp.int32) % (NP - N)) + N
    src_p = jnp.concatenate([edge_index[0], pad]).reshape(NW, NCH, CH)
    dst_p = jnp.concatenate([edge_index[1], pad]).reshape(NW, NCH, CH)
    edges = jnp.stack([src_p, dst_p], axis=2)
    ones_ch = jnp.ones((CH,), jnp.float32)
    z1 = jnp.zeros((NP,), jnp.float32)
    z2 = jnp.zeros((NP, 128), jnp.float32)
    x_pad = jnp.pad(x, ((0, NP - N), (0, 0)))

    degp = _deg_kernel(dst_p, ones_ch, z1)

    t1 = pl.pallas_call(
        _mm1_body,
        grid=(_GRID,),
        in_specs=[_row_spec(128), _whole_spec((128, 256))],
        out_specs=_row_spec(256),
        out_shape=jax.ShapeDtypeStruct((NP, 256), jnp.float32),
    )(x_pad, W1)

    d0 = degp[0].reshape(NP, 1)
    d1 = degp[1].reshape(NP, 1)

    s1a, s1b = pl.pallas_call(
        _scale_body,
        grid=(_GRID,),
        in_specs=[_row_spec(256), _row_spec(1), _row_spec(1)],
        out_specs=[_row_spec(128), _row_spec(128)],
        out_shape=[jax.ShapeDtypeStruct((NP, 128), jnp.float32)] * 2,
    )(t1, d0, d1)

    p1a = _agg128(s1a, edges, z2)
    p1b = _agg128(s1b, edges, z2)

    s2 = pl.pallas_call(
        _layer2_body,
        grid=(_GRID,),
        in_specs=[_part_spec(128), _part_spec(128), _row_spec(128),
                  _row_spec(128), _row_spec(1), _row_spec(1),
                  _whole_spec((1, 256)), _whole_spec((256, 128))],
        out_specs=_row_spec(128),
        out_shape=jax.ShapeDtypeStruct((NP, 128), jnp.float32),
    )(p1a, p1b, s1a, s1b, d0, d1, b1.reshape(1, 256), W2)

    p2 = _agg128(s2, edges, z2)

    out = pl.pallas_call(
        _final_body,
        grid=(_GRID,),
        in_specs=[_part_spec(128), _row_spec(128), _row_spec(1), _row_spec(1),
                  _whole_spec((1, 128))],
        out_specs=_row_spec(128),
        out_shape=jax.ShapeDtypeStruct((NP, 128), jnp.float32),
    )(p2, s2, d0, d1, b2.reshape(1, 128))

    return out[:N]


# speedup vs baseline: 1.1441x; 1.1441x over previous
import functools

import jax
import jax.numpy as jnp
from jax import lax
from jax.experimental import pallas as pl
from jax.experimental.pallas import tpu as pltpu
from jax.experimental.pallas import tpu_sc as plsc

N = 10000
E = 320000
NP = 10240
NC, NS = 2, 16
NW = NC * NS
NCH, CH = 128, 80
EPT = NCH * CH
E_PAD = NW * EPT
RPS = NP // NS

_MESH = plsc.VectorSubcoreMesh(
    core_axis_name="c", subcore_axis_name="s", num_cores=NC, num_subcores=NS
)


def _deg_body(dst_hbm, ones_hbm, z1_hbm, degp_hbm, dst_v, ones_v, acc):
    c = lax.axis_index("c")
    s = lax.axis_index("s")
    wid = s * NC + c
    r0 = s * RPS
    pltpu.sync_copy(z1_hbm.at[pl.ds(r0, RPS)], acc.at[pl.ds(r0, RPS)])
    pltpu.sync_copy(dst_hbm.at[wid], dst_v)
    pltpu.sync_copy(ones_hbm, ones_v)
    plsc.subcore_barrier()

    def step(j, carry):
        pltpu.sync_copy(ones_v, acc.at[dst_v.at[j]], add=True)
        return carry

    lax.fori_loop(0, NCH, step, 0)
    plsc.subcore_barrier()
    pltpu.sync_copy(acc.at[pl.ds(r0, RPS)], degp_hbm.at[c, pl.ds(r0, RPS)])


_deg_kernel = functools.partial(
    pl.kernel,
    _deg_body,
    out_type=jax.ShapeDtypeStruct((NC, NP), jnp.float32),
    mesh=_MESH,
    scratch_types=[
        pltpu.VMEM((NCH, CH), jnp.int32),
        pltpu.VMEM((CH,), jnp.float32),
        pltpu.VMEM_SHARED((NP,), jnp.float32),
    ],
)()


def _make_agg(tail, dtype):
    def body(table, edges_hbm, z_hbm, out, *scr):
        idxs, rows, isems, gsems, ssems, acc = \
            scr[0:4], scr[4:8], scr[8:12], scr[12:16], scr[16:20], scr[20]

        c = lax.axis_index("c")
        s = lax.axis_index("s")
        wid = s * NC + c
        r0 = s * RPS

        def idx_fetch(j, b):
            pltpu.async_copy(edges_hbm.at[wid, j], idxs[b], isems[b])

        def gather(j, b):
            pltpu.make_async_copy(edges_hbm.at[wid, j], idxs[b],
                                  isems[b]).wait()
            pltpu.async_copy(table.at[idxs[b].at[0]], rows[b], gsems[b])

        def scatter_wait(b):
            pltpu.make_async_copy(rows[b], acc.at[idxs[b].at[1]],
                                  ssems[b]).wait()

        pltpu.sync_copy(z_hbm.at[pl.ds(r0, RPS)], acc.at[pl.ds(r0, RPS)])
        idx_fetch(0, 0)
        idx_fetch(1, 1)
        gather(0, 0)
        plsc.subcore_barrier()

        def group(g, carry):
            for b in range(4):
                j = g * 4 + b

                @pl.when(j >= 2)
                def _():
                    scatter_wait((b + 2) % 4)

                @pl.when(j + 2 < NCH)
                def _():
                    idx_fetch(j + 2, (b + 2) % 4)

                @pl.when(j + 1 < NCH)
                def _():
                    gather(j + 1, (b + 1) % 4)

                pltpu.make_async_copy(table.at[idxs[b].at[0]], rows[b],
                                      gsems[b]).wait()
                pltpu.async_copy(rows[b], acc.at[idxs[b].at[1]], ssems[b],
                                 add=True)
            return carry

        lax.fori_loop(0, NCH // 4, group, 0)
        scatter_wait((NCH - 2) % 4)
        scatter_wait((NCH - 1) % 4)
        plsc.subcore_barrier()
        pltpu.sync_copy(acc.at[pl.ds(r0, RPS)], out.at[c, pl.ds(r0, RPS)])

    return functools.partial(
        pl.kernel,
        body,
        out_type=jax.ShapeDtypeStruct((NC, NP) + tail, dtype),
        mesh=_MESH,
        scratch_types=[
            *[pltpu.VMEM((2, CH), jnp.int32)] * 4,
            *[pltpu.VMEM((CH,) + tail, dtype)] * 4,
            *[pltpu.SemaphoreType.DMA] * 12,
            pltpu.VMEM_SHARED((NP,) + tail, dtype),
        ],
    )()


_agg128 = _make_agg((128,), jnp.float32)


_BM = 1280
_GRID = NP // _BM


def _mm1_body(x_ref, w_ref, o_ref):
    o_ref[...] = jnp.dot(x_ref[...], w_ref[...],
                         preferred_element_type=jnp.float32)


def _dinv(d0_ref, d1_ref):
    deg = d0_ref[...] + d1_ref[...] + 1.0
    return lax.rsqrt(jnp.maximum(deg, 1.0))


def _scale_body(t1_ref, d0_ref, d1_ref, a_ref, b_ref):
    dinv = _dinv(d0_ref, d1_ref)
    a_ref[...] = t1_ref[:, :128] * dinv
    b_ref[...] = t1_ref[:, 128:] * dinv


def _layer2_body(pa_ref, pb_ref, sa_ref, sb_ref, d0_ref, d1_ref, b1_ref,
                 w2_ref, o_ref):
    dinv = _dinv(d0_ref, d1_ref)
    ha = jax.nn.relu(dinv * (pa_ref[0] + pa_ref[1] + sa_ref[...])
                     + b1_ref[0:1, :128])
    hb = jax.nn.relu(dinv * (pb_ref[0] + pb_ref[1] + sb_ref[...])
                     + b1_ref[0:1, 128:])
    t2 = (jnp.dot(ha, w2_ref[:128, :], preferred_element_type=jnp.float32)
          + jnp.dot(hb, w2_ref[128:, :], preferred_element_type=jnp.float32))
    o_ref[...] = t2 * dinv


def _final_body(p_ref, s_ref, d0_ref, d1_ref, b2_ref, o_ref):
    agg = p_ref[0] + p_ref[1] + s_ref[...]
    o_ref[...] = _dinv(d0_ref, d1_ref) * agg + b2_ref[0:1, :]


def _row_spec(cols):
    return pl.BlockSpec((_BM, cols), lambda m: (m, 0))


def _whole_spec(shape):
    return pl.BlockSpec(shape, lambda m: tuple(0 for _ in shape))


def _part_spec(cols):
    return pl.BlockSpec((NC, _BM, cols), lambda m: (0, m, 0))


def kernel(x, edge_index, W1, b1, W2, b2):
    pad = (jnp.arange(E_PAD - E, dtype=jnp.int32) % (NP - N)) + N
    src_p = jnp.concatenate([edge_index[0], pad]).reshape(NW, NCH, CH)
    dst_p = jnp.concatenate([edge_index[1], pad]).reshape(NW, NCH, CH)
    edges = jnp.stack([src_p, dst_p], axis=2)
    ones_ch = jnp.ones((CH,), jnp.float32)
    z1 = jnp.zeros((NP,), jnp.float32)
    z2 = jnp.zeros((NP, 128), jnp.float32)
    x_pad = jnp.pad(x, ((0, NP - N), (0, 0)))

    degp = _deg_kernel(dst_p, ones_ch, z1)

    t1 = pl.pallas_call(
        _mm1_body,
        grid=(_GRID,),
        in_specs=[_row_spec(128), _whole_spec((128, 256))],
        out_specs=_row_spec(256),
        out_shape=jax.ShapeDtypeStruct((NP, 256), jnp.float32),
    )(x_pad, W1)

    d0 = degp[0].reshape(NP, 1)
    d1 = degp[1].reshape(NP, 1)

    s1a, s1b = pl.pallas_call(
        _scale_body,
        grid=(_GRID,),
        in_specs=[_row_spec(256), _row_spec(1), _row_spec(1)],
        out_specs=[_row_spec(128), _row_spec(128)],
        out_shape=[jax.ShapeDtypeStruct((NP, 128), jnp.float32)] * 2,
    )(t1, d0, d1)

    p1a = _agg128(s1a, edges, z2)
    p1b = _agg128(s1b, edges, z2)

    s2 = pl.pallas_call(
        _layer2_body,
        grid=(_GRID,),
        in_specs=[_part_spec(128), _part_spec(128), _row_spec(128),
                  _row_spec(128), _row_spec(1), _row_spec(1),
                  _whole_spec((1, 256)), _whole_spec((256, 128))],
        out_specs=_row_spec(128),
        out_shape=jax.ShapeDtypeStruct((NP, 128), jnp.float32),
    )(p1a, p1b, s1a, s1b, d0, d1, b1.reshape(1, 256), W2)

    p2 = _agg128(s2, edges, z2)

    out = pl.pallas_call(
        _final_body,
        grid=(_GRID,),
        in_specs=[_part_spec(128), _row_spec(128), _row_spec(1), _row_spec(1),
                  _whole_spec((1, 128))],
        out_specs=_row_spec(128),
        out_shape=jax.ShapeDtypeStruct((NP, 128), jnp.float32),
    )(p2, s2, d0, d1, b2.reshape(1, 128))

    return out[:N]


# speedup vs baseline: 1.1515x; 1.0065x over previous
import functools

import jax
import jax.numpy as jnp
from jax import lax
from jax.experimental import pallas as pl
from jax.experimental.pallas import tpu as pltpu
from jax.experimental.pallas import tpu_sc as plsc

N = 10000
E = 320000
NP = 10240
NC, NS = 2, 16
NW = NC * NS
NCH, CH = 128, 80
EPT = NCH * CH
E_PAD = NW * EPT
RPS = NP // NS

_MESH = plsc.VectorSubcoreMesh(
    core_axis_name="c", subcore_axis_name="s", num_cores=NC, num_subcores=NS
)


def _deg_body(dst_hbm, ones_hbm, z1_hbm, degp_hbm, dst_v, ones_v, acc, sem):
    c = lax.axis_index("c")
    s = lax.axis_index("s")
    wid = s * NC + c
    r0 = s * RPS
    pltpu.sync_copy(z1_hbm.at[pl.ds(r0, RPS)], acc.at[pl.ds(r0, RPS)])
    pltpu.sync_copy(dst_hbm.at[wid], dst_v)
    pltpu.sync_copy(ones_hbm, ones_v)
    plsc.subcore_barrier()

    def fire(j, carry):
        pltpu.async_copy(ones_v, acc.at[dst_v.at[j]], sem, add=True)
        return carry

    def drain(j, carry):
        pltpu.make_async_copy(ones_v, acc.at[dst_v.at[0]], sem).wait()
        return carry

    lax.fori_loop(0, NCH, fire, 0)
    lax.fori_loop(0, NCH, drain, 0)
    plsc.subcore_barrier()
    pltpu.sync_copy(acc.at[pl.ds(r0, RPS)], degp_hbm.at[c, pl.ds(r0, RPS)])


_deg_kernel = functools.partial(
    pl.kernel,
    _deg_body,
    out_type=jax.ShapeDtypeStruct((NC, NP), jnp.float32),
    mesh=_MESH,
    scratch_types=[
        pltpu.VMEM((NCH, CH), jnp.int32),
        pltpu.VMEM((CH,), jnp.float32),
        pltpu.VMEM_SHARED((NP,), jnp.float32),
        pltpu.SemaphoreType.DMA,
    ],
)()


def _make_agg(tail, dtype, num_tables):
    def body(*refs):
        tables = refs[:num_tables]
        edges_hbm, z_hbm = refs[num_tables:num_tables + 2]
        outs = refs[num_tables + 2:2 * num_tables + 2]
        scr = refs[2 * num_tables + 2:]
        idxs, rows, isems, gsems, ssems, acc = \
            scr[0:4], scr[4:8], scr[8:12], scr[12:16], scr[16:20], scr[20]

        c = lax.axis_index("c")
        s = lax.axis_index("s")
        wid = s * NC + c
        r0 = s * RPS

        def idx_fetch(j, b):
            pltpu.async_copy(edges_hbm.at[wid, j], idxs[b], isems[b])

        for h in range(num_tables):
            table = tables[h]

            def gather(j, b, table=table):
                pltpu.make_async_copy(edges_hbm.at[wid, j], idxs[b],
                                      isems[b]).wait()
                pltpu.async_copy(table.at[idxs[b].at[0]], rows[b], gsems[b])

            def scatter_wait(b):
                pltpu.make_async_copy(rows[b], acc.at[idxs[b].at[1]],
                                      ssems[b]).wait()

            pltpu.sync_copy(z_hbm.at[pl.ds(r0, RPS)], acc.at[pl.ds(r0, RPS)])
            idx_fetch(0, 0)
            idx_fetch(1, 1)
            gather(0, 0)
            plsc.subcore_barrier()

            def group(g, carry, gather=gather, scatter_wait=scatter_wait,
                      table=table):
                for b in range(4):
                    j = g * 4 + b

                    @pl.when(j >= 2)
                    def _():
                        scatter_wait((b + 2) % 4)

                    @pl.when(j + 2 < NCH)
                    def _():
                        idx_fetch(j + 2, (b + 2) % 4)

                    @pl.when(j + 1 < NCH)
                    def _():
                        gather(j + 1, (b + 1) % 4)

                    pltpu.make_async_copy(table.at[idxs[b].at[0]], rows[b],
                                          gsems[b]).wait()
                    pltpu.async_copy(rows[b], acc.at[idxs[b].at[1]], ssems[b],
                                     add=True)
                return carry

            lax.fori_loop(0, NCH // 4, group, 0)
            scatter_wait((NCH - 2) % 4)
            scatter_wait((NCH - 1) % 4)
            plsc.subcore_barrier()
            pltpu.sync_copy(acc.at[pl.ds(r0, RPS)],
                            outs[h].at[c, pl.ds(r0, RPS)])
            if h + 1 < num_tables:
                plsc.subcore_barrier()

    return functools.partial(
        pl.kernel,
        body,
        out_type=[jax.ShapeDtypeStruct((NC, NP) + tail, dtype)] * num_tables,
        mesh=_MESH,
        scratch_types=[
            *[pltpu.VMEM((2, CH), jnp.int32)] * 4,
            *[pltpu.VMEM((CH,) + tail, dtype)] * 4,
            *[pltpu.SemaphoreType.DMA] * 12,
            pltpu.VMEM_SHARED((NP,) + tail, dtype),
        ],
    )()


_agg_two = _make_agg((128,), jnp.float32, 2)
_agg_one = _make_agg((128,), jnp.float32, 1)


_BM = 1280
_GRID = NP // _BM


def _mm1_body(x_ref, w_ref, o_ref):
    o_ref[...] = jnp.dot(x_ref[...], w_ref[...],
                         preferred_element_type=jnp.float32)


def _dinv(d0_ref, d1_ref):
    deg = d0_ref[...] + d1_ref[...] + 1.0
    return lax.rsqrt(jnp.maximum(deg, 1.0))


def _scale_body(t1_ref, d0_ref, d1_ref, a_ref, b_ref):
    dinv = _dinv(d0_ref, d1_ref)
    a_ref[...] = t1_ref[:, :128] * dinv
    b_ref[...] = t1_ref[:, 128:] * dinv


def _layer2_body(pa_ref, pb_ref, sa_ref, sb_ref, d0_ref, d1_ref, b1_ref,
                 w2_ref, o_ref):
    dinv = _dinv(d0_ref, d1_ref)
    ha = jax.nn.relu(dinv * (pa_ref[0] + pa_ref[1] + sa_ref[...])
                     + b1_ref[0:1, :128])
    hb = jax.nn.relu(dinv * (pb_ref[0] + pb_ref[1] + sb_ref[...])
                     + b1_ref[0:1, 128:])
    t2 = (jnp.dot(ha, w2_ref[:128, :], preferred_element_type=jnp.float32)
          + jnp.dot(hb, w2_ref[128:, :], preferred_element_type=jnp.float32))
    o_ref[...] = t2 * dinv


def _final_body(p_ref, s_ref, d0_ref, d1_ref, b2_ref, o_ref):
    agg = p_ref[0] + p_ref[1] + s_ref[...]
    o_ref[...] = _dinv(d0_ref, d1_ref) * agg + b2_ref[0:1, :]


def _row_spec(cols):
    return pl.BlockSpec((_BM, cols), lambda m: (m, 0))


def _whole_spec(shape):
    return pl.BlockSpec(shape, lambda m: tuple(0 for _ in shape))


def _part_spec(cols):
    return pl.BlockSpec((NC, _BM, cols), lambda m: (0, m, 0))


def kernel(x, edge_index, W1, b1, W2, b2):
    pad = (jnp.arange(E_PAD - E, dtype=jnp.int32) % (NP - N)) + N
    src_p = jnp.concatenate([edge_index[0], pad]).reshape(NW, NCH, CH)
    dst_p = jnp.concatenate([edge_index[1], pad]).reshape(NW, NCH, CH)
    edges = jnp.stack([src_p, dst_p], axis=2)
    ones_ch = jnp.ones((CH,), jnp.float32)
    z1 = jnp.zeros((NP,), jnp.float32)
    z2 = jnp.zeros((NP, 128), jnp.float32)
    x_pad = jnp.pad(x, ((0, NP - N), (0, 0)))

    degp = _deg_kernel(dst_p, ones_ch, z1)

    t1 = pl.pallas_call(
        _mm1_body,
        grid=(_GRID,),
        in_specs=[_row_spec(128), _whole_spec((128, 256))],
        out_specs=_row_spec(256),
        out_shape=jax.ShapeDtypeStruct((NP, 256), jnp.float32),
    )(x_pad, W1)

    d0 = degp[0].reshape(NP, 1)
    d1 = degp[1].reshape(NP, 1)

    s1a, s1b = pl.pallas_call(
        _scale_body,
        grid=(_GRID,),
        in_specs=[_row_spec(256), _row_spec(1), _row_spec(1)],
        out_specs=[_row_spec(128), _row_spec(128)],
        out_shape=[jax.ShapeDtypeStruct((NP, 128), jnp.float32)] * 2,
    )(t1, d0, d1)

    p1a, p1b = _agg_two(s1a, s1b, edges, z2)

    s2 = pl.pallas_call(
        _layer2_body,
        grid=(_GRID,),
        in_specs=[_part_spec(128), _part_spec(128), _row_spec(128),
                  _row_spec(128), _row_spec(1), _row_spec(1),
                  _whole_spec((1, 256)), _whole_spec((256, 128))],
        out_specs=_row_spec(128),
        out_shape=jax.ShapeDtypeStruct((NP, 128), jnp.float32),
    )(p1a, p1b, s1a, s1b, d0, d1, b1.reshape(1, 256), W2)

    (p2,) = _agg_one(s2, edges, z2)

    out = pl.pallas_call(
        _final_body,
        grid=(_GRID,),
        in_specs=[_part_spec(128), _row_spec(128), _row_spec(1), _row_spec(1),
                  _whole_spec((1, 128))],
        out_specs=_row_spec(128),
        out_shape=jax.ShapeDtypeStruct((NP, 128), jnp.float32),
    )(p2, s2, d0, d1, b2.reshape(1, 128))

    return out[:N]


# speedup vs baseline: 1.2048x; 1.0463x over previous
import functools

import jax
import jax.numpy as jnp
from jax import lax
from jax.experimental import pallas as pl
from jax.experimental.pallas import tpu as pltpu
from jax.experimental.pallas import tpu_sc as plsc

N = 10000
E = 320000
NP = 10240
NC, NS = 2, 16
NW = NC * NS
NCH, CH = 128, 80
EPT = NCH * CH
E_PAD = NW * EPT
RPS = NP // NS

_MESH = plsc.VectorSubcoreMesh(
    core_axis_name="c", subcore_axis_name="s", num_cores=NC, num_subcores=NS
)


def _deg_body(dst_hbm, ones_hbm, z1_hbm, degp_hbm, dst_v, ones_v, acc, sem):
    c = lax.axis_index("c")
    s = lax.axis_index("s")
    wid = s * NC + c
    r0 = s * RPS
    pltpu.sync_copy(z1_hbm.at[pl.ds(r0, RPS)], acc.at[pl.ds(r0, RPS)])
    pltpu.sync_copy(dst_hbm.at[wid], dst_v)
    pltpu.sync_copy(ones_hbm, ones_v)
    plsc.subcore_barrier()

    def fire(j, carry):
        pltpu.async_copy(ones_v, acc.at[dst_v.at[j]], sem, add=True)
        return carry

    def drain(j, carry):
        pltpu.make_async_copy(ones_v, acc.at[dst_v.at[0]], sem).wait()
        return carry

    lax.fori_loop(0, NCH, fire, 0)
    lax.fori_loop(0, NCH, drain, 0)
    plsc.subcore_barrier()
    pltpu.sync_copy(acc.at[pl.ds(r0, RPS)], degp_hbm.at[c, pl.ds(r0, RPS)])


_deg_kernel = functools.partial(
    pl.kernel,
    _deg_body,
    out_type=jax.ShapeDtypeStruct((NC, NP), jnp.float32),
    mesh=_MESH,
    scratch_types=[
        pltpu.VMEM((NCH, CH), jnp.int32),
        pltpu.VMEM((CH,), jnp.float32),
        pltpu.VMEM_SHARED((NP,), jnp.float32),
        pltpu.SemaphoreType.DMA,
    ],
)()


def _make_agg(tail, dtype, num_tables):
    def body(*refs):
        tables = refs[:num_tables]
        edges_hbm, z_hbm = refs[num_tables:num_tables + 2]
        outs = refs[num_tables + 2:2 * num_tables + 2]
        scr = refs[2 * num_tables + 2:]
        idxs, rows, isems, gsems, ssems, acc = \
            scr[0:4], scr[4:8], scr[8:12], scr[12:16], scr[16:20], scr[20]

        c = lax.axis_index("c")
        s = lax.axis_index("s")
        wid = s * NC + c
        r0 = s * RPS

        def idx_fetch(j, b):
            pltpu.async_copy(edges_hbm.at[wid, j], idxs[b], isems[b])

        for h in range(num_tables):
            table = tables[h]

            def gather(j, b, table=table):
                pltpu.make_async_copy(edges_hbm.at[wid, j], idxs[b],
                                      isems[b]).wait()
                pltpu.async_copy(table.at[idxs[b].at[0]], rows[b], gsems[b])

            def scatter_wait(b):
                pltpu.make_async_copy(rows[b], acc.at[idxs[b].at[1]],
                                      ssems[b]).wait()

            pltpu.sync_copy(z_hbm.at[pl.ds(r0, RPS)], acc.at[pl.ds(r0, RPS)])
            idx_fetch(0, 0)
            idx_fetch(1, 1)
            gather(0, 0)
            plsc.subcore_barrier()

            def group(g, carry, gather=gather, scatter_wait=scatter_wait,
                      table=table):
                for b in range(4):
                    j = g * 4 + b

                    @pl.when(j >= 2)
                    def _():
                        scatter_wait((b + 2) % 4)

                    @pl.when(j + 2 < NCH)
                    def _():
                        idx_fetch(j + 2, (b + 2) % 4)

                    @pl.when(j + 1 < NCH)
                    def _():
                        gather(j + 1, (b + 1) % 4)

                    pltpu.make_async_copy(table.at[idxs[b].at[0]], rows[b],
                                          gsems[b]).wait()
                    pltpu.async_copy(rows[b], acc.at[idxs[b].at[1]], ssems[b],
                                     add=True)
                return carry

            lax.fori_loop(0, NCH // 4, group, 0)
            scatter_wait((NCH - 2) % 4)
            scatter_wait((NCH - 1) % 4)
            plsc.subcore_barrier()
            pltpu.sync_copy(acc.at[pl.ds(r0, RPS)],
                            outs[h].at[c, pl.ds(r0, RPS)])
            if h + 1 < num_tables:
                plsc.subcore_barrier()

    return functools.partial(
        pl.kernel,
        body,
        out_type=[jax.ShapeDtypeStruct((NC, NP) + tail, dtype)] * num_tables,
        mesh=_MESH,
        scratch_types=[
            *[pltpu.VMEM((2, CH), jnp.int32)] * 4,
            *[pltpu.VMEM((CH,) + tail, dtype)] * 4,
            *[pltpu.SemaphoreType.DMA] * 12,
            pltpu.VMEM_SHARED((NP,) + tail, dtype),
        ],
    )()


_agg_one = _make_agg((128,), jnp.float32, 1)

NCH1 = NCH * NC


def _agg_l1_body(tables_hbm, edges_hbm, z_hbm, out, *scr):
    idxs, rows, isems, gsems, ssems, acc = \
        scr[0:4], scr[4:8], scr[8:12], scr[12:16], scr[16:20], scr[20]

    c = lax.axis_index("c")
    s = lax.axis_index("s")
    r0 = s * RPS
    table = tables_hbm.at[c]

    def idx_fetch(j, b):
        pltpu.async_copy(edges_hbm.at[s, j], idxs[b], isems[b])

    def gather(j, b):
        pltpu.make_async_copy(edges_hbm.at[s, j], idxs[b], isems[b]).wait()
        pltpu.async_copy(table.at[idxs[b].at[0]], rows[b], gsems[b])

    def scatter_wait(b):
        pltpu.make_async_copy(rows[b], acc.at[idxs[b].at[1]], ssems[b]).wait()

    pltpu.sync_copy(z_hbm.at[pl.ds(r0, RPS)], acc.at[pl.ds(r0, RPS)])
    idx_fetch(0, 0)
    idx_fetch(1, 1)
    gather(0, 0)
    plsc.subcore_barrier()

    def group(g, carry):
        for b in range(4):
            j = g * 4 + b

            @pl.when(j >= 2)
            def _():
                scatter_wait((b + 2) % 4)

            @pl.when(j + 2 < NCH1)
            def _():
                idx_fetch(j + 2, (b + 2) % 4)

            @pl.when(j + 1 < NCH1)
            def _():
                gather(j + 1, (b + 1) % 4)

            pltpu.make_async_copy(table.at[idxs[b].at[0]], rows[b],
                                  gsems[b]).wait()
            pltpu.async_copy(rows[b], acc.at[idxs[b].at[1]], ssems[b],
                             add=True)
        return carry

    lax.fori_loop(0, NCH1 // 4, group, 0)
    scatter_wait((NCH1 - 2) % 4)
    scatter_wait((NCH1 - 1) % 4)
    plsc.subcore_barrier()
    pltpu.sync_copy(acc.at[pl.ds(r0, RPS)], out.at[c, pl.ds(r0, RPS)])


_agg_l1 = functools.partial(
    pl.kernel,
    _agg_l1_body,
    out_type=jax.ShapeDtypeStruct((NC, NP, 128), jnp.float32),
    mesh=_MESH,
    scratch_types=[
        *[pltpu.VMEM((2, CH), jnp.int32)] * 4,
        *[pltpu.VMEM((CH, 128), jnp.float32)] * 4,
        *[pltpu.SemaphoreType.DMA] * 12,
        pltpu.VMEM_SHARED((NP, 128), jnp.float32),
    ],
)()


_BM = 1280
_GRID = NP // _BM


def _mm1_body(x_ref, w_ref, o_ref):
    o_ref[...] = jnp.dot(x_ref[...], w_ref[...],
                         preferred_element_type=jnp.float32)


def _dinv(d0_ref, d1_ref):
    deg = d0_ref[...] + d1_ref[...] + 1.0
    return lax.rsqrt(jnp.maximum(deg, 1.0))


def _scale_body(t1_ref, d0_ref, d1_ref, s_ref):
    dinv = _dinv(d0_ref, d1_ref)
    s_ref[0] = t1_ref[:, :128] * dinv
    s_ref[1] = t1_ref[:, 128:] * dinv


def _layer2_body(p1_ref, s1_ref, d0_ref, d1_ref, b1_ref,
                 w2_ref, o_ref):
    dinv = _dinv(d0_ref, d1_ref)
    ha = jax.nn.relu(dinv * (p1_ref[0] + s1_ref[0]) + b1_ref[0:1, :128])
    hb = jax.nn.relu(dinv * (p1_ref[1] + s1_ref[1]) + b1_ref[0:1, 128:])
    t2 = (jnp.dot(ha, w2_ref[:128, :], preferred_element_type=jnp.float32)
          + jnp.dot(hb, w2_ref[128:, :], preferred_element_type=jnp.float32))
    o_ref[...] = t2 * dinv


def _final_body(p_ref, s_ref, d0_ref, d1_ref, b2_ref, o_ref):
    agg = p_ref[0] + p_ref[1] + s_ref[...]
    o_ref[...] = _dinv(d0_ref, d1_ref) * agg + b2_ref[0:1, :]


def _row_spec(cols):
    return pl.BlockSpec((_BM, cols), lambda m: (m, 0))


def _whole_spec(shape):
    return pl.BlockSpec(shape, lambda m: tuple(0 for _ in shape))


def _part_spec(cols):
    return pl.BlockSpec((NC, _BM, cols), lambda m: (0, m, 0))


def kernel(x, edge_index, W1, b1, W2, b2):
    pad = (jnp.arange(E_PAD - E, dtype=jnp.int32) % (NP - N)) + N
    src_p = jnp.concatenate([edge_index[0], pad]).reshape(NW, NCH, CH)
    dst_p = jnp.concatenate([edge_index[1], pad]).reshape(NW, NCH, CH)
    edges = jnp.stack([src_p, dst_p], axis=2)
    ones_ch = jnp.ones((CH,), jnp.float32)
    z1 = jnp.zeros((NP,), jnp.float32)
    z2 = jnp.zeros((NP, 128), jnp.float32)
    x_pad = jnp.pad(x, ((0, NP - N), (0, 0)))

    degp = _deg_kernel(dst_p, ones_ch, z1)

    t1 = pl.pallas_call(
        _mm1_body,
        grid=(_GRID,),
        in_specs=[_row_spec(128), _whole_spec((128, 256))],
        out_specs=_row_spec(256),
        out_shape=jax.ShapeDtypeStruct((NP, 256), jnp.float32),
    )(x_pad, W1)

    d0 = degp[0].reshape(NP, 1)
    d1 = degp[1].reshape(NP, 1)

    s1 = pl.pallas_call(
        _scale_body,
        grid=(_GRID,),
        in_specs=[_row_spec(256), _row_spec(1), _row_spec(1)],
        out_specs=_part_spec(128),
        out_shape=jax.ShapeDtypeStruct((2, NP, 128), jnp.float32),
    )(t1, d0, d1)

    p1 = _agg_l1(s1, edges.reshape(NS, NCH1, 2, CH), z2)

    s2 = pl.pallas_call(
        _layer2_body,
        grid=(_GRID,),
        in_specs=[_part_spec(128), _part_spec(128), _row_spec(1), _row_spec(1),
                  _whole_spec((1, 256)), _whole_spec((256, 128))],
        out_specs=_row_spec(128),
        out_shape=jax.ShapeDtypeStruct((NP, 128), jnp.float32),
    )(p1, s1, d0, d1, b1.reshape(1, 256), W2)

    (p2,) = _agg_one(s2, edges, z2)

    out = pl.pallas_call(
        _final_body,
        grid=(_GRID,),
        in_specs=[_part_spec(128), _row_spec(128), _row_spec(1), _row_spec(1),
                  _whole_spec((1, 128))],
        out_specs=_row_spec(128),
        out_shape=jax.ShapeDtypeStruct((NP, 128), jnp.float32),
    )(p2, s2, d0, d1, b2.reshape(1, 128))

    return out[:N]
